# baseline (speedup 1.0000x reference)
import jax
import jax.numpy as jnp
import numpy as np
from jax.experimental import pallas as pl
from jax.experimental.pallas import tpu as pltpu

NUM_TIMESTEPS = 1000
BETA_START = 1e-4
BETA_END = 0.02

SAMPLES_PER_BLOCK = 8

_BETAS = np.linspace(BETA_START, BETA_END, NUM_TIMESTEPS, dtype=np.float64)
_AC = np.cumprod(1.0 - _BETAS)
_SQRT_AC = np.sqrt(_AC).astype(np.float32)
_SQRT_1MAC = np.sqrt(1.0 - _AC).astype(np.float32)


def _f32_cumprod_tables():
    betas = np.linspace(BETA_START, BETA_END, NUM_TIMESTEPS, dtype=np.float32)
    ac = np.cumprod((1.0 - betas).astype(np.float32), dtype=np.float32)
    sqrt_ac = np.sqrt(ac).astype(np.float32)
    sqrt_1mac = np.sqrt((1.0 - ac).astype(np.float32)).astype(np.float32)
    return sqrt_ac, sqrt_1mac


def _body(ts_ref, ac_ref, mac_ref, x_ref, n_ref, o_ref):
    g = pl.program_id(0)
    base = g * SAMPLES_PER_BLOCK
    a_s = []
    c_s = []
    for i in range(SAMPLES_PER_BLOCK):
        t = ts_ref[base + i]
        a_s.append(ac_ref[t])
        c_s.append(mac_ref[t])
    a_vec = jnp.stack(a_s).reshape(SAMPLES_PER_BLOCK, 1, 1, 1)
    c_vec = jnp.stack(c_s).reshape(SAMPLES_PER_BLOCK, 1, 1, 1)
    o_ref[...] = a_vec * x_ref[...] + c_vec * n_ref[...]


def kernel(original_samples, noise, timesteps):
    B, C, H, W = original_samples.shape
    S = SAMPLES_PER_BLOCK
    sqrt_ac, sqrt_1mac = _f32_cumprod_tables()
    ts = timesteps.astype(jnp.int32)

    grid_spec = pltpu.PrefetchScalarGridSpec(
        num_scalar_prefetch=3,
        grid=(B // S,),
        in_specs=[
            pl.BlockSpec((S, C, H, W), lambda b, *_: (b, 0, 0, 0)),
            pl.BlockSpec((S, C, H, W), lambda b, *_: (b, 0, 0, 0)),
        ],
        out_specs=pl.BlockSpec((S, C, H, W), lambda b, *_: (b, 0, 0, 0)),
    )
    out = pl.pallas_call(
        _body,
        grid_spec=grid_spec,
        out_shape=jax.ShapeDtypeStruct((B, C, H, W), jnp.float32),
        compiler_params=pltpu.CompilerParams(
            dimension_semantics=("arbitrary",),
        ),
    )(ts, jnp.asarray(sqrt_ac), jnp.asarray(sqrt_1mac), original_samples, noise)
    return out


# speedup vs baseline: 4.2877x; 4.2877x over previous
import jax
import jax.numpy as jnp
import numpy as np
from jax.experimental import pallas as pl
from jax.experimental.pallas import tpu as pltpu

NUM_TIMESTEPS = 1000
BETA_START = 1e-4
BETA_END = 0.02

ROWS_PER_BLOCK = 32


def _f32_cumprod_tables():
    betas = np.linspace(BETA_START, BETA_END, NUM_TIMESTEPS, dtype=np.float32)
    ac = np.cumprod((1.0 - betas).astype(np.float32), dtype=np.float32)
    sqrt_ac = np.sqrt(ac).astype(np.float32)
    sqrt_1mac = np.sqrt((1.0 - ac).astype(np.float32)).astype(np.float32)
    return sqrt_ac, sqrt_1mac


def _body(a_ref, c_ref, x_ref, n_ref, o_ref):
    o_ref[...] = a_ref[...] * x_ref[...] + c_ref[...] * n_ref[...]


def kernel(original_samples, noise, timesteps):
    B, C, H, W = original_samples.shape
    R = C * H
    Rb = ROWS_PER_BLOCK
    x = jnp.transpose(original_samples, (1, 2, 3, 0)).reshape(R, W, B)
    n = jnp.transpose(noise, (1, 2, 3, 0)).reshape(R, W, B)
    sqrt_ac, sqrt_1mac = _f32_cumprod_tables()
    a_vec = jnp.take(jnp.asarray(sqrt_ac), timesteps, axis=0).reshape(1, 1, B)
    c_vec = jnp.take(jnp.asarray(sqrt_1mac), timesteps, axis=0).reshape(1, 1, B)

    out = pl.pallas_call(
        _body,
        grid=(R // Rb,),
        in_specs=[
            pl.BlockSpec((1, 1, B), lambda i: (0, 0, 0)),
            pl.BlockSpec((1, 1, B), lambda i: (0, 0, 0)),
            pl.BlockSpec((Rb, W, B), lambda i: (i, 0, 0)),
            pl.BlockSpec((Rb, W, B), lambda i: (i, 0, 0)),
        ],
        out_specs=pl.BlockSpec((Rb, W, B), lambda i: (i, 0, 0)),
        out_shape=jax.ShapeDtypeStruct((R, W, B), jnp.float32),
        compiler_params=pltpu.CompilerParams(
            dimension_semantics=("arbitrary",),
        ),
    )(a_vec, c_vec, x, n)
    return jnp.transpose(out.reshape(C, H, W, B), (3, 0, 1, 2))


# speedup vs baseline: 4.4610x; 1.0404x over previous
import jax
import jax.numpy as jnp
import numpy as np
from jax import lax
from jax.experimental import pallas as pl
from jax.experimental.pallas import tpu as pltpu

NUM_TIMESTEPS = 1000
BETA_START = 1e-4
BETA_END = 0.02

ROWS_PER_BLOCK = 32
_TPAD = 1024


def _f32_cumprod_tables():
    betas = np.linspace(BETA_START, BETA_END, NUM_TIMESTEPS, dtype=np.float32)
    ac = np.cumprod((1.0 - betas).astype(np.float32), dtype=np.float32)
    sqrt_ac = np.sqrt(ac).astype(np.float32)
    sqrt_1mac = np.sqrt((1.0 - ac).astype(np.float32)).astype(np.float32)
    tabs = np.zeros((2, _TPAD), dtype=np.float32)
    tabs[0, :NUM_TIMESTEPS] = sqrt_ac
    tabs[1, :NUM_TIMESTEPS] = sqrt_1mac
    return tabs


def _body(ts_ref, tab_ref, x_ref, n_ref, o_ref, scale_ref):
    @pl.when(pl.program_id(0) == 0)
    def _gather_scales():
        ts = ts_ref[...]
        iota = lax.broadcasted_iota(jnp.int32, (_TPAD, ts.shape[1]), 0)
        onehot = (iota == ts).astype(jnp.float32)
        scale_ref[...] = jnp.dot(
            tab_ref[...], onehot, preferred_element_type=jnp.float32
        )

    a = scale_ref[0:1, :][None]
    c = scale_ref[1:2, :][None]
    o_ref[...] = a * x_ref[...] + c * n_ref[...]


def kernel(original_samples, noise, timesteps):
    B, C, H, W = original_samples.shape
    R = C * H
    Rb = ROWS_PER_BLOCK
    x = jnp.transpose(original_samples, (1, 2, 3, 0)).reshape(R, W, B)
    n = jnp.transpose(noise, (1, 2, 3, 0)).reshape(R, W, B)
    tabs = jnp.asarray(_f32_cumprod_tables())
    ts = timesteps.astype(jnp.int32).reshape(1, B)

    out = pl.pallas_call(
        _body,
        grid=(R // Rb,),
        in_specs=[
            pl.BlockSpec((1, B), lambda i: (0, 0)),
            pl.BlockSpec((2, _TPAD), lambda i: (0, 0)),
            pl.BlockSpec((Rb, W, B), lambda i: (i, 0, 0)),
            pl.BlockSpec((Rb, W, B), lambda i: (i, 0, 0)),
        ],
        out_specs=pl.BlockSpec((Rb, W, B), lambda i: (i, 0, 0)),
        out_shape=jax.ShapeDtypeStruct((R, W, B), jnp.float32),
        scratch_shapes=[pltpu.VMEM((2, B), jnp.float32)],
        compiler_params=pltpu.CompilerParams(
            dimension_semantics=("arbitrary",),
        ),
    )(ts, tabs, x, n)
    return jnp.transpose(out.reshape(C, H, W, B), (3, 0, 1, 2))
